# baseline (speedup 1.0000x reference)
import functools

import jax
import jax.numpy as jnp
from jax import lax
from jax.experimental import pallas as pl
from jax.experimental.pallas import tpu as pltpu
from jax.experimental.pallas import tpu_sc as plsc

B, C, N, K, OUT = 16, 64, 1024, 20, 128
EPS = 1e-5
NW = 32
ROWS_PER_W = B * N // NW


def _prep_body(x_ref, w_ref, d_ref, pt_ref, qt_ref):
    xb = x_ref[0]
    wa = w_ref[:, :C]
    wq = w_ref[:, C:] - wa
    xx = jnp.sum(xb * xb, axis=0, keepdims=True)
    g = lax.dot_general(xb, xb, (((0,), (0,)), ((), ())),
                        preferred_element_type=jnp.float32)
    d_ref[0] = 2.0 * g - xx
    pt_ref[0] = lax.dot_general(xb, wa, (((0,), (1,)), ((), ())),
                                preferred_element_type=jnp.float32)
    qt_ref[0] = lax.dot_general(xb, wq, (((0,), (1,)), ((), ())),
                                preferred_element_type=jnp.float32)


def _prep(x, W):
    return pl.pallas_call(
        _prep_body,
        grid=(B,),
        in_specs=[
            pl.BlockSpec((1, C, N), lambda b: (b, 0, 0)),
            pl.BlockSpec((OUT, 2 * C), lambda b: (0, 0)),
        ],
        out_specs=[
            pl.BlockSpec((1, N, N), lambda b: (b, 0, 0)),
            pl.BlockSpec((1, N, OUT), lambda b: (b, 0, 0)),
            pl.BlockSpec((1, N, OUT), lambda b: (b, 0, 0)),
        ],
        out_shape=[
            jax.ShapeDtypeStruct((B, N, N), jnp.float32),
            jax.ShapeDtypeStruct((B, N, OUT), jnp.float32),
            jax.ShapeDtypeStruct((B, N, OUT), jnp.float32),
        ],
    )(x, W)


def _sort16(kk, vv):
    return plsc.sort_key_val(kk, vv)


def _merge16_to_32(a, bq):
    ak, ai = a
    bk, bi = bq
    brk = lax.rev(bk, (0,))
    bri = lax.rev(bi, (0,))
    m = ak <= brk
    lok = jnp.where(m, ak, brk)
    loi = jnp.where(m, ai, bri)
    hik = jnp.where(m, brk, ak)
    hii = jnp.where(m, bri, ai)
    lok, loi = _sort16(lok, loi)
    hik, hii = _sort16(hik, hii)
    return (lok, hik, loi, hii)


def _merge32_top32(A, Bq):
    a0, a1, ai0, ai1 = A
    b0, b1, bi0, bi1 = Bq
    rb1 = lax.rev(b1, (0,))
    rbi1 = lax.rev(bi1, (0,))
    rb0 = lax.rev(b0, (0,))
    rbi0 = lax.rev(bi0, (0,))
    m0 = a0 >= rb1
    c0 = jnp.where(m0, a0, rb1)
    ci0 = jnp.where(m0, ai0, rbi1)
    m1 = a1 >= rb0
    c1 = jnp.where(m1, a1, rb0)
    ci1 = jnp.where(m1, ai1, rbi0)
    m = c0 <= c1
    d0 = jnp.where(m, c0, c1)
    e0 = jnp.where(m, ci0, ci1)
    d1 = jnp.where(m, c1, c0)
    e1 = jnp.where(m, ci1, ci0)
    d0, e0 = _sort16(d0, e0)
    d1, e1 = _sort16(d1, e1)
    return (d0, d1, e0, e1)


def _select(d, ptf):
    mesh = plsc.VectorSubcoreMesh(core_axis_name="c", subcore_axis_name="s")

    @functools.partial(
        pl.kernel, mesh=mesh,
        compiler_params=pltpu.CompilerParams(needs_layout_passes=False),
        out_type=[
            jax.ShapeDtypeStruct((B, N, OUT), jnp.float32),
            jax.ShapeDtypeStruct((B, N, OUT), jnp.float32),
            jax.ShapeDtypeStruct((B, N, OUT), jnp.float32),
            jax.ShapeDtypeStruct((NW, N), jnp.int32),
        ],
        scratch_types=[
            pltpu.VMEM((2, 4, N), jnp.float32),
            pltpu.VMEM((N,), jnp.int32),
            pltpu.VMEM((2, 112), jnp.int32),
            pltpu.VMEM((2, 96, OUT), jnp.float32),
            pltpu.VMEM((2, 4, OUT), jnp.float32),
            pltpu.VMEM((2, 4, OUT), jnp.float32),
            pltpu.VMEM((2, 4, OUT), jnp.float32),
            pltpu.VMEM((N,), jnp.int32),
            pltpu.VMEM_SHARED((8 * N, OUT), jnp.float32),
            pltpu.SemaphoreType.DMA,
            pltpu.SemaphoreType.DMA,
            pltpu.SemaphoreType.DMA,
        ],
    )
    def sel(d_hbm, pt_hbm, smin_hbm, smax_hbm, ssum_hbm, cnt_hbm,
            d_v, iota_v, idx_v, g_v, smin_v, smax_v, ssum_v, cnt_v,
            pt_sh, d_sem, g_sem, o_sem):
        c = lax.axis_index("c")
        sid = lax.axis_index("s")
        wid = c * 16 + sid
        b = wid // 2
        n0 = (wid % 2) * (N // 2)
        pltpu.sync_copy(pt_hbm.at[pl.ds(c * (8 * N) + sid * 512, 512)],
                        pt_sh.at[pl.ds(sid * 512, 512)])
        plsc.subcore_barrier()
        lane = lax.broadcasted_iota(jnp.int32, (16,), 0)
        zero16 = jnp.zeros((16,), jnp.int32)
        ones16 = jnp.ones((16,), jnp.int32)
        hi_mask = lane >= 12
        for r in range(N // 16):
            iota_v[pl.ds(r * 16, 16)] = lane + r * 16
            cnt_v[pl.ds(r * 16, 16)] = zero16
        base = (b % 8) * N
        for s in range(2):
            for u in range(4):
                idx_v[s, pl.ds(u * 24 + 16, 16)] = base + lane + u * 16
        T = ROWS_PER_W // 4

        def d_copy(t):
            return pltpu.make_async_copy(d_hbm.at[b, pl.ds(n0 + 4 * t, 4)],
                                         d_v.at[t % 2], d_sem)

        def g_copy(t):
            return pltpu.make_async_copy(pt_sh.at[idx_v.at[t % 2, pl.ds(0, 96)]],
                                         g_v.at[t % 2], g_sem)

        def o_copies(t):
            nd = pl.ds(n0 + 4 * t, 4)
            s = t % 2
            return (
                pltpu.make_async_copy(smin_v.at[s], smin_hbm.at[b, nd], o_sem),
                pltpu.make_async_copy(smax_v.at[s], smax_hbm.at[b, nd], o_sem),
                pltpu.make_async_copy(ssum_v.at[s], ssum_hbm.at[b, nd], o_sem),
            )

        def topk(t, u):
            s = t % 2
            stack = []
            for r in range(0, N // 16, 2):
                a = _sort16(d_v[s, u, pl.ds(r * 16, 16)],
                            iota_v[pl.ds(r * 16, 16)])
                bq = _sort16(d_v[s, u, pl.ds((r + 1) * 16, 16)],
                             iota_v[pl.ds((r + 1) * 16, 16)])
                cur = _merge16_to_32(a, bq)
                lvl = 0
                while stack and stack[-1][0] == lvl:
                    cur = _merge32_top32(stack.pop()[1], cur)
                    lvl += 1
                stack.append((lvl, cur))
            d0, d1, e0, e1 = stack[0][1]
            idx_v[s, pl.ds(u * 24, 16)] = e1 + base
            plsc.store_compressed(idx_v.at[s, pl.ds(u * 24 + 16, 16)],
                                  e0 + base, mask=hi_mask)
            plsc.addupdate_scatter(cnt_v, [e0], ones16, mask=hi_mask)
            plsc.addupdate_scatter(cnt_v, [e1], ones16)

        def reduce(t, u):
            s = t % 2
            ng = OUT // 16
            gr = g_v.at[s]
            r0 = u * 24
            mn = [gr[r0, pl.ds(gi * 16, 16)] for gi in range(ng)]
            mx = list(mn)
            sm = list(mn)
            for j in range(1, 20):
                for gi in range(ng):
                    v = gr[r0 + j, pl.ds(gi * 16, 16)]
                    mn[gi] = jnp.minimum(mn[gi], v)
                    mx[gi] = jnp.maximum(mx[gi], v)
                    sm[gi] = sm[gi] + v
            for gi in range(ng):
                sl = pl.ds(gi * 16, 16)
                smin_v[s, u, sl] = mn[gi]
                smax_v[s, u, sl] = mx[gi]
                ssum_v[s, u, sl] = sm[gi]

        d_copy(0).start()

        def body(t, carry):
            @pl.when(t < T)
            def _():
                d_copy(t).wait()

                @pl.when(t + 1 < T)
                def _():
                    d_copy(t + 1).start()

                for u in range(4):
                    topk(t, u)
                g_copy(t).start()

            jb = t - 1

            @pl.when(jb >= 0)
            def _():
                g_copy(jb).wait()

                @pl.when(jb >= 2)
                def _():
                    for cp in o_copies(jb - 2):
                        cp.wait()

                for u in range(4):
                    reduce(jb, u)
                for cp in o_copies(jb):
                    cp.start()
            return carry

        lax.fori_loop(0, T + 1, body, 0)
        for t in (T - 2, T - 1):
            for cp in o_copies(t):
                cp.wait()
        pltpu.sync_copy(cnt_v, cnt_hbm.at[wid])

    return sel(d, ptf)


def _stats_body(ssum_ref, qt_ref, pt_ref, cnt_ref, o_ref):
    bb = pl.program_id(0)

    @pl.when(bb == 0)
    def _init():
        o_ref[...] = jnp.zeros_like(o_ref)

    s = ssum_ref[0]
    q = qt_ref[0]
    p = pt_ref[0]
    c = cnt_ref[0]
    o_ref[0, :] += jnp.sum(s, axis=0)
    o_ref[1, :] += jnp.sum(q, axis=0)
    o_ref[2, :] += jnp.sum(q * q, axis=0)
    o_ref[3, :] += jnp.sum(q * s, axis=0)
    o_ref[4, :] += jnp.sum(c * p * p, axis=0)


def _stats(ssum, qt, pt, cntf):
    return pl.pallas_call(
        _stats_body,
        grid=(B,),
        in_specs=[
            pl.BlockSpec((1, N, OUT), lambda b: (b, 0, 0)),
            pl.BlockSpec((1, N, OUT), lambda b: (b, 0, 0)),
            pl.BlockSpec((1, N, OUT), lambda b: (b, 0, 0)),
            pl.BlockSpec((1, N, 1), lambda b: (b, 0, 0)),
        ],
        out_specs=pl.BlockSpec((8, OUT), lambda b: (0, 0)),
        out_shape=jax.ShapeDtypeStruct((8, OUT), jnp.float32),
    )(ssum, qt, pt, cntf)


def _gelu(z):
    return 0.5 * z * (1.0 + lax.erf(z * 0.7071067811865476))


def _final_body(smin_ref, smax_ref, qt_ref, sc_ref, sh_ref, o_ref):
    q = qt_ref[0]
    sc = sc_ref[0, :]
    sh = sh_ref[0, :]
    z1 = (smin_ref[0] + q) * sc + sh
    z2 = (smax_ref[0] + q) * sc + sh
    o_ref[0] = jnp.transpose(jnp.maximum(_gelu(z1), _gelu(z2)))


def _final(smin, smax, qt, scale, shift):
    return pl.pallas_call(
        _final_body,
        grid=(B,),
        in_specs=[
            pl.BlockSpec((1, N, OUT), lambda b: (b, 0, 0)),
            pl.BlockSpec((1, N, OUT), lambda b: (b, 0, 0)),
            pl.BlockSpec((1, N, OUT), lambda b: (b, 0, 0)),
            pl.BlockSpec((1, OUT), lambda b: (0, 0)),
            pl.BlockSpec((1, OUT), lambda b: (0, 0)),
        ],
        out_specs=pl.BlockSpec((1, OUT, N), lambda b: (b, 0, 0)),
        out_shape=jax.ShapeDtypeStruct((B, OUT, N), jnp.float32),
    )(smin, smax, qt, scale, shift)


def kernel(x, W, gamma, beta):
    d, pt, qt = _prep(x, W)
    smin, smax, ssum, cnt = _select(d, pt.reshape(B * N, OUT))
    cntf = cnt.reshape(B, 2, N).sum(axis=1).astype(jnp.float32).reshape(B, N, 1)
    sums = _stats(ssum, qt, pt, cntf)
    m = float(B * N * K)
    mean = (sums[0] + K * sums[1]) / m
    ey2 = (sums[4] + 2.0 * sums[3] + K * sums[2]) / m
    var = ey2 - mean * mean
    scale = gamma / jnp.sqrt(var + EPS)
    shift = beta - mean * scale
    return _final(smin, smax, qt, scale.reshape(1, OUT), shift.reshape(1, OUT))


# speedup vs baseline: 1.5485x; 1.5485x over previous
import functools

import jax
import jax.numpy as jnp
from jax import lax
from jax.experimental import pallas as pl
from jax.experimental.pallas import tpu as pltpu
from jax.experimental.pallas import tpu_sc as plsc

B, C, N, K, OUT = 16, 64, 1024, 20, 128
EPS = 1e-5
NW = 32
ROWS_PER_W = B * N // NW


def _prep_body(x_ref, w_ref, d_ref, pt_ref, qt_ref):
    xb = x_ref[0]
    wa = w_ref[:, :C]
    wq = w_ref[:, C:] - wa
    xx = jnp.sum(xb * xb, axis=0, keepdims=True)
    g = lax.dot_general(xb, xb, (((0,), (0,)), ((), ())),
                        preferred_element_type=jnp.float32)
    d_ref[0] = 2.0 * g - xx
    pt_ref[0] = lax.dot_general(xb, wa, (((0,), (1,)), ((), ())),
                                preferred_element_type=jnp.float32)
    qt_ref[0] = lax.dot_general(xb, wq, (((0,), (1,)), ((), ())),
                                preferred_element_type=jnp.float32)


def _prep(x, W):
    return pl.pallas_call(
        _prep_body,
        grid=(B,),
        in_specs=[
            pl.BlockSpec((1, C, N), lambda b: (b, 0, 0)),
            pl.BlockSpec((OUT, 2 * C), lambda b: (0, 0)),
        ],
        out_specs=[
            pl.BlockSpec((1, N, N), lambda b: (b, 0, 0)),
            pl.BlockSpec((1, N, OUT), lambda b: (b, 0, 0)),
            pl.BlockSpec((1, N, OUT), lambda b: (b, 0, 0)),
        ],
        out_shape=[
            jax.ShapeDtypeStruct((B, N, N), jnp.float32),
            jax.ShapeDtypeStruct((B, N, OUT), jnp.float32),
            jax.ShapeDtypeStruct((B, N, OUT), jnp.float32),
        ],
    )(x, W)


def _sort16(kk, vv):
    return plsc.sort_key_val(kk, vv)


def _merge16_to_32(a, bq):
    ak, ai = a
    bk, bi = bq
    brk = lax.rev(bk, (0,))
    bri = lax.rev(bi, (0,))
    m = ak <= brk
    lok = jnp.where(m, ak, brk)
    loi = jnp.where(m, ai, bri)
    hik = jnp.where(m, brk, ak)
    hii = jnp.where(m, bri, ai)
    lok, loi = _sort16(lok, loi)
    hik, hii = _sort16(hik, hii)
    return (lok, hik, loi, hii)


def _merge32_top32(A, Bq):
    a0, a1, ai0, ai1 = A
    b0, b1, bi0, bi1 = Bq
    rb1 = lax.rev(b1, (0,))
    rbi1 = lax.rev(bi1, (0,))
    rb0 = lax.rev(b0, (0,))
    rbi0 = lax.rev(bi0, (0,))
    m0 = a0 >= rb1
    c0 = jnp.where(m0, a0, rb1)
    ci0 = jnp.where(m0, ai0, rbi1)
    m1 = a1 >= rb0
    c1 = jnp.where(m1, a1, rb0)
    ci1 = jnp.where(m1, ai1, rbi0)
    m = c0 <= c1
    d0 = jnp.where(m, c0, c1)
    e0 = jnp.where(m, ci0, ci1)
    d1 = jnp.where(m, c1, c0)
    e1 = jnp.where(m, ci1, ci0)
    d0, e0 = _sort16(d0, e0)
    d1, e1 = _sort16(d1, e1)
    return (d0, d1, e0, e1)


def _select(d, ptf):
    mesh = plsc.VectorSubcoreMesh(core_axis_name="c", subcore_axis_name="s")

    @functools.partial(
        pl.kernel, mesh=mesh,
        compiler_params=pltpu.CompilerParams(needs_layout_passes=False),
        out_type=[
            jax.ShapeDtypeStruct((B, N, OUT), jnp.float32),
            jax.ShapeDtypeStruct((B, N, OUT), jnp.float32),
            jax.ShapeDtypeStruct((B, N, OUT), jnp.float32),
            jax.ShapeDtypeStruct((NW, N), jnp.int32),
        ],
        scratch_types=[
            pltpu.VMEM((4, N), jnp.float32),
            pltpu.VMEM((N,), jnp.int32),
            pltpu.VMEM((4, 32), jnp.int32),
            pltpu.VMEM((4, 24, OUT), jnp.float32),
            pltpu.VMEM((4, OUT), jnp.float32),
            pltpu.VMEM((4, OUT), jnp.float32),
            pltpu.VMEM((4, OUT), jnp.float32),
            pltpu.VMEM((N,), jnp.int32),
            pltpu.SemaphoreType.DMA,
            pltpu.SemaphoreType.DMA,
            pltpu.SemaphoreType.DMA,
            pltpu.SemaphoreType.DMA,
        ],
    )
    def sel(d_hbm, pt_hbm, smin_hbm, smax_hbm, ssum_hbm, cnt_hbm,
            d_v, iota_v, idx_v, g_v, smin_v, smax_v, ssum_v, cnt_v,
            d_sem, g_sem, g2_sem, o_sem):
        wid = lax.axis_index("s") * 2 + lax.axis_index("c")
        b = wid // 2
        n0 = (wid % 2) * (N // 2)
        lane = lax.broadcasted_iota(jnp.int32, (16,), 0)
        zero16 = jnp.zeros((16,), jnp.int32)
        ones16 = jnp.ones((16,), jnp.int32)
        hi_mask = lane >= 12
        for r in range(N // 16):
            iota_v[pl.ds(r * 16, 16)] = lane + r * 16
            cnt_v[pl.ds(r * 16, 16)] = zero16
        base = b * N
        for s in range(4):
            idx_v[s, pl.ds(16, 16)] = base + lane + s * 16
        R = ROWS_PER_W

        def d_copy(i):
            return pltpu.make_async_copy(d_hbm.at[b, n0 + i],
                                         d_v.at[i % 4], d_sem)

        def g_copy(i):
            s = i % 4
            return (
                pltpu.make_async_copy(pt_hbm.at[idx_v.at[s, pl.ds(0, 16)]],
                                      g_v.at[s, pl.ds(0, 16)], g_sem),
                pltpu.make_async_copy(pt_hbm.at[idx_v.at[s, pl.ds(16, 8)]],
                                      g_v.at[s, pl.ds(16, 8)], g2_sem),
            )

        def o_copies(i):
            n = n0 + i
            s = i % 4
            return (
                pltpu.make_async_copy(smin_v.at[s], smin_hbm.at[b, n], o_sem),
                pltpu.make_async_copy(smax_v.at[s], smax_hbm.at[b, n], o_sem),
                pltpu.make_async_copy(ssum_v.at[s], ssum_hbm.at[b, n], o_sem),
            )

        def topk(i):
            s = i % 4
            stack = []
            for r in range(0, N // 16, 2):
                a = _sort16(d_v[s, pl.ds(r * 16, 16)],
                            iota_v[pl.ds(r * 16, 16)])
                bq = _sort16(d_v[s, pl.ds((r + 1) * 16, 16)],
                             iota_v[pl.ds((r + 1) * 16, 16)])
                cur = _merge16_to_32(a, bq)
                lvl = 0
                while stack and stack[-1][0] == lvl:
                    cur = _merge32_top32(stack.pop()[1], cur)
                    lvl += 1
                stack.append((lvl, cur))
            d0, d1, e0, e1 = stack[0][1]
            idx_v[s, pl.ds(0, 16)] = e1 + base
            plsc.store_compressed(idx_v.at[s, pl.ds(16, 16)], e0 + base,
                                  mask=hi_mask)
            plsc.addupdate_scatter(cnt_v, [e0], ones16, mask=hi_mask)
            plsc.addupdate_scatter(cnt_v, [e1], ones16)

        def reduce(i):
            s = i % 4
            o = i % 4
            ng = OUT // 16
            gr = g_v.at[s]
            mn = [gr[0, pl.ds(gi * 16, 16)] for gi in range(ng)]
            mx = list(mn)
            sm = list(mn)
            for j in range(1, 20):
                for gi in range(ng):
                    v = gr[j, pl.ds(gi * 16, 16)]
                    mn[gi] = jnp.minimum(mn[gi], v)
                    mx[gi] = jnp.maximum(mx[gi], v)
                    sm[gi] = sm[gi] + v
            for gi in range(ng):
                sl = pl.ds(gi * 16, 16)
                smin_v[o, sl] = mn[gi]
                smax_v[o, sl] = mx[gi]
                ssum_v[o, sl] = sm[gi]

        d_copy(0).start()
        d_copy(1).start()
        d_copy(2).start()

        def body(i, carry):
            @pl.when(i < R)
            def _():
                d_copy(i).wait()

                @pl.when(i + 3 < R)
                def _():
                    d_copy(i + 3).start()

                topk(i)
                for cp in g_copy(i):
                    cp.start()

            j = i - 3

            @pl.when(j >= 0)
            def _():
                for cp in g_copy(j):
                    cp.wait()

                @pl.when(j >= 4)
                def _():
                    for cp in o_copies(j - 4):
                        cp.wait()

                reduce(j)
                for cp in o_copies(j):
                    cp.start()
            return carry

        lax.fori_loop(0, R + 3, body, 0)
        for j in (R - 4, R - 3, R - 2, R - 1):
            for cp in o_copies(j):
                cp.wait()
        pltpu.sync_copy(cnt_v, cnt_hbm.at[wid])

    return sel(d, ptf)


def _stats_body(ssum_ref, qt_ref, pt_ref, cnt_ref, o_ref):
    bb = pl.program_id(0)

    @pl.when(bb == 0)
    def _init():
        o_ref[...] = jnp.zeros_like(o_ref)

    s = ssum_ref[0]
    q = qt_ref[0]
    p = pt_ref[0]
    c = cnt_ref[0]
    o_ref[0, :] += jnp.sum(s, axis=0)
    o_ref[1, :] += jnp.sum(q, axis=0)
    o_ref[2, :] += jnp.sum(q * q, axis=0)
    o_ref[3, :] += jnp.sum(q * s, axis=0)
    o_ref[4, :] += jnp.sum(c * p * p, axis=0)


def _stats(ssum, qt, pt, cntf):
    return pl.pallas_call(
        _stats_body,
        grid=(B,),
        in_specs=[
            pl.BlockSpec((1, N, OUT), lambda b: (b, 0, 0)),
            pl.BlockSpec((1, N, OUT), lambda b: (b, 0, 0)),
            pl.BlockSpec((1, N, OUT), lambda b: (b, 0, 0)),
            pl.BlockSpec((1, N, 1), lambda b: (b, 0, 0)),
        ],
        out_specs=pl.BlockSpec((8, OUT), lambda b: (0, 0)),
        out_shape=jax.ShapeDtypeStruct((8, OUT), jnp.float32),
    )(ssum, qt, pt, cntf)


def _gelu(z):
    return 0.5 * z * (1.0 + lax.erf(z * 0.7071067811865476))


def _final_body(smin_ref, smax_ref, qt_ref, sc_ref, sh_ref, o_ref):
    q = qt_ref[0]
    sc = sc_ref[0, :]
    sh = sh_ref[0, :]
    z1 = (smin_ref[0] + q) * sc + sh
    z2 = (smax_ref[0] + q) * sc + sh
    o_ref[0] = jnp.transpose(jnp.maximum(_gelu(z1), _gelu(z2)))


def _final(smin, smax, qt, scale, shift):
    return pl.pallas_call(
        _final_body,
        grid=(B,),
        in_specs=[
            pl.BlockSpec((1, N, OUT), lambda b: (b, 0, 0)),
            pl.BlockSpec((1, N, OUT), lambda b: (b, 0, 0)),
            pl.BlockSpec((1, N, OUT), lambda b: (b, 0, 0)),
            pl.BlockSpec((1, OUT), lambda b: (0, 0)),
            pl.BlockSpec((1, OUT), lambda b: (0, 0)),
        ],
        out_specs=pl.BlockSpec((1, OUT, N), lambda b: (b, 0, 0)),
        out_shape=jax.ShapeDtypeStruct((B, OUT, N), jnp.float32),
    )(smin, smax, qt, scale, shift)


def kernel(x, W, gamma, beta):
    d, pt, qt = _prep(x, W)
    smin, smax, ssum, cnt = _select(d, pt.reshape(B * N, OUT))
    cntf = cnt.reshape(B, 2, N).sum(axis=1).astype(jnp.float32).reshape(B, N, 1)
    sums = _stats(ssum, qt, pt, cntf)
    m = float(B * N * K)
    mean = (sums[0] + K * sums[1]) / m
    ey2 = (sums[4] + 2.0 * sums[3] + K * sums[2]) / m
    var = ey2 - mean * mean
    scale = gamma / jnp.sqrt(var + EPS)
    shift = beta - mean * scale
    return _final(smin, smax, qt, scale.reshape(1, OUT), shift.reshape(1, OUT))


# speedup vs baseline: 1.5641x; 1.0101x over previous
import functools

import jax
import jax.numpy as jnp
from jax import lax
from jax.experimental import pallas as pl
from jax.experimental.pallas import tpu as pltpu
from jax.experimental.pallas import tpu_sc as plsc

B, C, N, K, OUT = 16, 64, 1024, 20, 128
EPS = 1e-5
NW = 32
ROWS_PER_W = B * N // NW


def _prep_body(x_ref, w_ref, d_ref, pt_ref, qt_ref):
    xb = x_ref[0]
    wa = w_ref[:, :C]
    wq = w_ref[:, C:] - wa
    xx = jnp.sum(xb * xb, axis=0, keepdims=True)
    g = lax.dot_general(xb, xb, (((0,), (0,)), ((), ())),
                        preferred_element_type=jnp.float32)
    d_ref[0] = 2.0 * g - xx
    pt_ref[0] = lax.dot_general(xb, wa, (((0,), (1,)), ((), ())),
                                preferred_element_type=jnp.float32)
    qt_ref[0] = lax.dot_general(xb, wq, (((0,), (1,)), ((), ())),
                                preferred_element_type=jnp.float32)


def _prep(x, W):
    return pl.pallas_call(
        _prep_body,
        grid=(B,),
        in_specs=[
            pl.BlockSpec((1, C, N), lambda b: (b, 0, 0)),
            pl.BlockSpec((OUT, 2 * C), lambda b: (0, 0)),
        ],
        out_specs=[
            pl.BlockSpec((1, N, N), lambda b: (b, 0, 0)),
            pl.BlockSpec((1, N, OUT), lambda b: (b, 0, 0)),
            pl.BlockSpec((1, N, OUT), lambda b: (b, 0, 0)),
        ],
        out_shape=[
            jax.ShapeDtypeStruct((B, N, N), jnp.float32),
            jax.ShapeDtypeStruct((B, N, OUT), jnp.float32),
            jax.ShapeDtypeStruct((B, N, OUT), jnp.float32),
        ],
    )(x, W)


def _sort16(kk, vv):
    return plsc.sort_key_val(kk, vv)


def _merge16_to_32(a, bq):
    ak, ai = a
    bk, bi = bq
    brk = lax.rev(bk, (0,))
    bri = lax.rev(bi, (0,))
    m = ak <= brk
    lok = jnp.where(m, ak, brk)
    loi = jnp.where(m, ai, bri)
    hik = jnp.where(m, brk, ak)
    hii = jnp.where(m, bri, ai)
    lok, loi = _sort16(lok, loi)
    hik, hii = _sort16(hik, hii)
    return (lok, hik, loi, hii)


def _merge32_top32(A, Bq):
    a0, a1, ai0, ai1 = A
    b0, b1, bi0, bi1 = Bq
    rb1 = lax.rev(b1, (0,))
    rbi1 = lax.rev(bi1, (0,))
    rb0 = lax.rev(b0, (0,))
    rbi0 = lax.rev(bi0, (0,))
    m0 = a0 >= rb1
    c0 = jnp.where(m0, a0, rb1)
    ci0 = jnp.where(m0, ai0, rbi1)
    m1 = a1 >= rb0
    c1 = jnp.where(m1, a1, rb0)
    ci1 = jnp.where(m1, ai1, rbi0)
    m = c0 <= c1
    d0 = jnp.where(m, c0, c1)
    e0 = jnp.where(m, ci0, ci1)
    d1 = jnp.where(m, c1, c0)
    e1 = jnp.where(m, ci1, ci0)
    d0, e0 = _sort16(d0, e0)
    d1, e1 = _sort16(d1, e1)
    return (d0, d1, e0, e1)


def _select(d, ptf):
    mesh = plsc.VectorSubcoreMesh(core_axis_name="c", subcore_axis_name="s")

    @functools.partial(
        pl.kernel, mesh=mesh,
        compiler_params=pltpu.CompilerParams(needs_layout_passes=False),
        out_type=[
            jax.ShapeDtypeStruct((B, N, OUT), jnp.float32),
            jax.ShapeDtypeStruct((B, N, OUT), jnp.float32),
            jax.ShapeDtypeStruct((B, N, OUT), jnp.float32),
            jax.ShapeDtypeStruct((NW, N), jnp.int32),
        ],
        scratch_types=[
            pltpu.VMEM((2, 8, N), jnp.float32),
            pltpu.VMEM((N,), jnp.int32),
            pltpu.VMEM((4, 32), jnp.int32),
            pltpu.VMEM((4, 24, OUT), jnp.float32),
            pltpu.VMEM((2, 8, OUT), jnp.float32),
            pltpu.VMEM((2, 8, OUT), jnp.float32),
            pltpu.VMEM((2, 8, OUT), jnp.float32),
            pltpu.VMEM((N,), jnp.int32),
            pltpu.SemaphoreType.DMA,
            pltpu.SemaphoreType.DMA,
            pltpu.SemaphoreType.DMA,
            pltpu.SemaphoreType.DMA,
        ],
    )
    def sel(d_hbm, pt_hbm, smin_hbm, smax_hbm, ssum_hbm, cnt_hbm,
            d_v, iota_v, idx_v, g_v, smin_v, smax_v, ssum_v, cnt_v,
            d_sem, g_sem, g2_sem, o_sem):
        wid = lax.axis_index("s") * 2 + lax.axis_index("c")
        b = wid // 2
        n0 = (wid % 2) * (N // 2)
        lane = lax.broadcasted_iota(jnp.int32, (16,), 0)
        zero16 = jnp.zeros((16,), jnp.int32)
        ones16 = jnp.ones((16,), jnp.int32)
        hi_mask = lane >= 12
        for r in range(N // 16):
            iota_v[pl.ds(r * 16, 16)] = lane + r * 16
            cnt_v[pl.ds(r * 16, 16)] = zero16
        base = b * N
        for s in range(4):
            idx_v[s, pl.ds(16, 16)] = base + lane + s * 16
        R = ROWS_PER_W

        def d_copy(tb):
            return pltpu.make_async_copy(d_hbm.at[b, pl.ds(n0 + 8 * tb, 8)],
                                         d_v.at[tb % 2], d_sem)

        def g_copy(i):
            s = i % 4
            return (
                pltpu.make_async_copy(pt_hbm.at[idx_v.at[s, pl.ds(0, 16)]],
                                      g_v.at[s, pl.ds(0, 16)], g_sem),
                pltpu.make_async_copy(pt_hbm.at[idx_v.at[s, pl.ds(16, 8)]],
                                      g_v.at[s, pl.ds(16, 8)], g2_sem),
            )

        def o_copies(tb):
            nd = pl.ds(n0 + 8 * tb, 8)
            s = tb % 2
            return (
                pltpu.make_async_copy(smin_v.at[s], smin_hbm.at[b, nd], o_sem),
                pltpu.make_async_copy(smax_v.at[s], smax_hbm.at[b, nd], o_sem),
                pltpu.make_async_copy(ssum_v.at[s], ssum_hbm.at[b, nd], o_sem),
            )

        def topk(i):
            s = i % 4
            db = (i // 8) % 2
            du = i % 8
            stack = []
            for r in range(0, N // 16, 2):
                a = _sort16(d_v[db, du, pl.ds(r * 16, 16)],
                            iota_v[pl.ds(r * 16, 16)])
                bq = _sort16(d_v[db, du, pl.ds((r + 1) * 16, 16)],
                             iota_v[pl.ds((r + 1) * 16, 16)])
                cur = _merge16_to_32(a, bq)
                lvl = 0
                while stack and stack[-1][0] == lvl:
                    cur = _merge32_top32(stack.pop()[1], cur)
                    lvl += 1
                stack.append((lvl, cur))
            d0, d1, e0, e1 = stack[0][1]
            idx_v[s, pl.ds(0, 16)] = e1 + base
            plsc.store_compressed(idx_v.at[s, pl.ds(16, 16)], e0 + base,
                                  mask=hi_mask)
            plsc.addupdate_scatter(cnt_v, [e0], ones16, mask=hi_mask)
            plsc.addupdate_scatter(cnt_v, [e1], ones16)

        def reduce(i):
            s = i % 4
            o = i % 4
            ng = OUT // 16
            gr = g_v.at[s]
            mn = [gr[0, pl.ds(gi * 16, 16)] for gi in range(ng)]
            mx = list(mn)
            sm = list(mn)
            for j in range(1, 20):
                for gi in range(ng):
                    v = gr[j, pl.ds(gi * 16, 16)]
                    mn[gi] = jnp.minimum(mn[gi], v)
                    mx[gi] = jnp.maximum(mx[gi], v)
                    sm[gi] = sm[gi] + v
            ob = (i // 8) % 2
            ou = i % 8
            for gi in range(ng):
                sl = pl.ds(gi * 16, 16)
                smin_v[ob, ou, sl] = mn[gi]
                smax_v[ob, ou, sl] = mx[gi]
                ssum_v[ob, ou, sl] = sm[gi]

        TB = R // 8
        d_copy(0).start()

        def body(i, carry):
            @pl.when(i < R)
            def _():
                @pl.when(i % 8 == 0)
                def _():
                    d_copy(i // 8).wait()

                    @pl.when(i // 8 + 1 < TB)
                    def _():
                        d_copy(i // 8 + 1).start()

                topk(i)
                for cp in g_copy(i):
                    cp.start()

            j = i - 3

            @pl.when(j >= 0)
            def _():
                for cp in g_copy(j):
                    cp.wait()

                @pl.when((j % 8 == 0) & (j >= 16))
                def _():
                    for cp in o_copies(j // 8 - 2):
                        cp.wait()

                reduce(j)

                @pl.when(j % 8 == 7)
                def _():
                    for cp in o_copies(j // 8):
                        cp.start()
            return carry

        lax.fori_loop(0, R + 3, body, 0)
        for tb in (TB - 2, TB - 1):
            for cp in o_copies(tb):
                cp.wait()
        pltpu.sync_copy(cnt_v, cnt_hbm.at[wid])

    return sel(d, ptf)


def _stats_body(ssum_ref, qt_ref, pt_ref, cnt_ref, o_ref):
    bb = pl.program_id(0)

    @pl.when(bb == 0)
    def _init():
        o_ref[...] = jnp.zeros_like(o_ref)

    s = ssum_ref[0]
    q = qt_ref[0]
    p = pt_ref[0]
    c = cnt_ref[0]
    o_ref[0, :] += jnp.sum(s, axis=0)
    o_ref[1, :] += jnp.sum(q, axis=0)
    o_ref[2, :] += jnp.sum(q * q, axis=0)
    o_ref[3, :] += jnp.sum(q * s, axis=0)
    o_ref[4, :] += jnp.sum(c * p * p, axis=0)


def _stats(ssum, qt, pt, cntf):
    return pl.pallas_call(
        _stats_body,
        grid=(B,),
        in_specs=[
            pl.BlockSpec((1, N, OUT), lambda b: (b, 0, 0)),
            pl.BlockSpec((1, N, OUT), lambda b: (b, 0, 0)),
            pl.BlockSpec((1, N, OUT), lambda b: (b, 0, 0)),
            pl.BlockSpec((1, N, 1), lambda b: (b, 0, 0)),
        ],
        out_specs=pl.BlockSpec((8, OUT), lambda b: (0, 0)),
        out_shape=jax.ShapeDtypeStruct((8, OUT), jnp.float32),
    )(ssum, qt, pt, cntf)


def _gelu(z):
    return 0.5 * z * (1.0 + lax.erf(z * 0.7071067811865476))


def _final_body(smin_ref, smax_ref, qt_ref, sc_ref, sh_ref, o_ref):
    q = qt_ref[0]
    sc = sc_ref[0, :]
    sh = sh_ref[0, :]
    z1 = (smin_ref[0] + q) * sc + sh
    z2 = (smax_ref[0] + q) * sc + sh
    o_ref[0] = jnp.transpose(jnp.maximum(_gelu(z1), _gelu(z2)))


def _final(smin, smax, qt, scale, shift):
    return pl.pallas_call(
        _final_body,
        grid=(B,),
        in_specs=[
            pl.BlockSpec((1, N, OUT), lambda b: (b, 0, 0)),
            pl.BlockSpec((1, N, OUT), lambda b: (b, 0, 0)),
            pl.BlockSpec((1, N, OUT), lambda b: (b, 0, 0)),
            pl.BlockSpec((1, OUT), lambda b: (0, 0)),
            pl.BlockSpec((1, OUT), lambda b: (0, 0)),
        ],
        out_specs=pl.BlockSpec((1, OUT, N), lambda b: (b, 0, 0)),
        out_shape=jax.ShapeDtypeStruct((B, OUT, N), jnp.float32),
    )(smin, smax, qt, scale, shift)


def kernel(x, W, gamma, beta):
    d, pt, qt = _prep(x, W)
    smin, smax, ssum, cnt = _select(d, pt.reshape(B * N, OUT))
    cntf = cnt.reshape(B, 2, N).sum(axis=1).astype(jnp.float32).reshape(B, N, 1)
    sums = _stats(ssum, qt, pt, cntf)
    m = float(B * N * K)
    mean = (sums[0] + K * sums[1]) / m
    ey2 = (sums[4] + 2.0 * sums[3] + K * sums[2]) / m
    var = ey2 - mean * mean
    scale = gamma / jnp.sqrt(var + EPS)
    shift = beta - mean * scale
    return _final(smin, smax, qt, scale.reshape(1, OUT), shift.reshape(1, OUT))


# speedup vs baseline: 1.5713x; 1.0046x over previous
import functools

import jax
import jax.numpy as jnp
from jax import lax
from jax.experimental import pallas as pl
from jax.experimental.pallas import tpu as pltpu
from jax.experimental.pallas import tpu_sc as plsc

B, C, N, K, OUT = 16, 64, 1024, 20, 128
EPS = 1e-5
NW = 32
ROWS_PER_W = B * N // NW


def _prep_body(x_ref, w_ref, d_ref, pt_ref, qt_ref):
    xb = x_ref[0]
    wa = w_ref[:, :C]
    wq = w_ref[:, C:] - wa
    xx = jnp.sum(xb * xb, axis=0, keepdims=True)
    g = lax.dot_general(xb, xb, (((0,), (0,)), ((), ())),
                        preferred_element_type=jnp.float32)
    d_ref[0] = 2.0 * g - xx
    pt_ref[0] = lax.dot_general(xb, wa, (((0,), (1,)), ((), ())),
                                preferred_element_type=jnp.float32)
    qt_ref[0] = lax.dot_general(xb, wq, (((0,), (1,)), ((), ())),
                                preferred_element_type=jnp.float32)


def _prep(x, W):
    return pl.pallas_call(
        _prep_body,
        grid=(B,),
        in_specs=[
            pl.BlockSpec((1, C, N), lambda b: (b, 0, 0)),
            pl.BlockSpec((OUT, 2 * C), lambda b: (0, 0)),
        ],
        out_specs=[
            pl.BlockSpec((1, N, N), lambda b: (b, 0, 0)),
            pl.BlockSpec((1, N, OUT), lambda b: (b, 0, 0)),
            pl.BlockSpec((1, N, OUT), lambda b: (b, 0, 0)),
        ],
        out_shape=[
            jax.ShapeDtypeStruct((B, N, N), jnp.float32),
            jax.ShapeDtypeStruct((B, N, OUT), jnp.float32),
            jax.ShapeDtypeStruct((B, N, OUT), jnp.float32),
        ],
    )(x, W)


def _sort16(kk, vv):
    return plsc.sort_key_val(kk, vv)


def _merge16_to_32(a, bq):
    ak, ai = a
    bk, bi = bq
    brk = lax.rev(bk, (0,))
    bri = lax.rev(bi, (0,))
    m = ak <= brk
    lok = jnp.where(m, ak, brk)
    loi = jnp.where(m, ai, bri)
    hik = jnp.where(m, brk, ak)
    hii = jnp.where(m, bri, ai)
    lok, loi = _sort16(lok, loi)
    hik, hii = _sort16(hik, hii)
    return (lok, hik, loi, hii)


def _merge32_top32(A, Bq):
    a0, a1, ai0, ai1 = A
    b0, b1, bi0, bi1 = Bq
    rb1 = lax.rev(b1, (0,))
    rbi1 = lax.rev(bi1, (0,))
    rb0 = lax.rev(b0, (0,))
    rbi0 = lax.rev(bi0, (0,))
    m0 = a0 >= rb1
    c0 = jnp.where(m0, a0, rb1)
    ci0 = jnp.where(m0, ai0, rbi1)
    m1 = a1 >= rb0
    c1 = jnp.where(m1, a1, rb0)
    ci1 = jnp.where(m1, ai1, rbi0)
    m = c0 <= c1
    d0 = jnp.where(m, c0, c1)
    e0 = jnp.where(m, ci0, ci1)
    d1 = jnp.where(m, c1, c0)
    e1 = jnp.where(m, ci1, ci0)
    d0, e0 = _sort16(d0, e0)
    d1, e1 = _sort16(d1, e1)
    return (d0, d1, e0, e1)


def _select(d, ptf):
    mesh = plsc.VectorSubcoreMesh(core_axis_name="c", subcore_axis_name="s")

    @functools.partial(
        pl.kernel, mesh=mesh,
        compiler_params=pltpu.CompilerParams(needs_layout_passes=False),
        out_type=[
            jax.ShapeDtypeStruct((B, N, OUT), jnp.float32),
            jax.ShapeDtypeStruct((B, N, OUT), jnp.float32),
            jax.ShapeDtypeStruct((B, N, OUT), jnp.float32),
            jax.ShapeDtypeStruct((NW, N), jnp.int32),
        ],
        scratch_types=[
            pltpu.VMEM((2, 8, N), jnp.float32),
            pltpu.VMEM((N,), jnp.int32),
            pltpu.VMEM((2, 56), jnp.int32),
            pltpu.VMEM((2, 48, OUT), jnp.float32),
            pltpu.VMEM((2, 8, OUT), jnp.float32),
            pltpu.VMEM((2, 8, OUT), jnp.float32),
            pltpu.VMEM((2, 8, OUT), jnp.float32),
            pltpu.VMEM((N,), jnp.int32),
            pltpu.SemaphoreType.DMA,
            pltpu.SemaphoreType.DMA,
            pltpu.SemaphoreType.DMA,
        ],
    )
    def sel(d_hbm, pt_hbm, smin_hbm, smax_hbm, ssum_hbm, cnt_hbm,
            d_v, iota_v, idx_v, g_v, smin_v, smax_v, ssum_v, cnt_v,
            d_sem, g_sem, o_sem):
        wid = lax.axis_index("s") * 2 + lax.axis_index("c")
        b = wid // 2
        n0 = (wid % 2) * (N // 2)
        lane = lax.broadcasted_iota(jnp.int32, (16,), 0)
        zero16 = jnp.zeros((16,), jnp.int32)
        ones16 = jnp.ones((16,), jnp.int32)
        hi_mask = lane >= 12
        for r in range(N // 16):
            iota_v[pl.ds(r * 16, 16)] = lane + r * 16
            cnt_v[pl.ds(r * 16, 16)] = zero16
        base = b * N
        for s in range(2):
            for u in range(2):
                idx_v[s, pl.ds(u * 24 + 16, 16)] = base + lane + u * 16
        R = ROWS_PER_W

        def d_copy(tb):
            return pltpu.make_async_copy(d_hbm.at[b, pl.ds(n0 + 8 * tb, 8)],
                                         d_v.at[tb % 2], d_sem)

        def g_copy(gb):
            return pltpu.make_async_copy(pt_hbm.at[idx_v.at[gb % 2, pl.ds(0, 48)]],
                                         g_v.at[gb % 2], g_sem)

        def o_copies(tb):
            nd = pl.ds(n0 + 8 * tb, 8)
            s = tb % 2
            return (
                pltpu.make_async_copy(smin_v.at[s], smin_hbm.at[b, nd], o_sem),
                pltpu.make_async_copy(smax_v.at[s], smax_hbm.at[b, nd], o_sem),
                pltpu.make_async_copy(ssum_v.at[s], ssum_hbm.at[b, nd], o_sem),
            )

        def topk(i):
            s = i % 4
            db = (i // 8) % 2
            du = i % 8
            stack = []
            for r in range(0, N // 16, 2):
                a = _sort16(d_v[db, du, pl.ds(r * 16, 16)],
                            iota_v[pl.ds(r * 16, 16)])
                bq = _sort16(d_v[db, du, pl.ds((r + 1) * 16, 16)],
                             iota_v[pl.ds((r + 1) * 16, 16)])
                cur = _merge16_to_32(a, bq)
                lvl = 0
                while stack and stack[-1][0] == lvl:
                    cur = _merge32_top32(stack.pop()[1], cur)
                    lvl += 1
                stack.append((lvl, cur))
            d0, d1, e0, e1 = stack[0][1]
            u24 = (i % 2) * 24
            gs = (i // 2) % 2
            idx_v[gs, pl.ds(u24, 16)] = e1 + base
            plsc.store_compressed(idx_v.at[gs, pl.ds(u24 + 16, 16)], e0 + base,
                                  mask=hi_mask)
            plsc.addupdate_scatter(cnt_v, [e0], ones16, mask=hi_mask)
            plsc.addupdate_scatter(cnt_v, [e1], ones16)

        def reduce(i):
            ng = OUT // 16
            gr = g_v.at[(i // 2) % 2]
            r0 = (i % 2) * 24
            mn = [gr[r0, pl.ds(gi * 16, 16)] for gi in range(ng)]
            mx = list(mn)
            sm = list(mn)
            for j in range(1, 20):
                for gi in range(ng):
                    v = gr[r0 + j, pl.ds(gi * 16, 16)]
                    mn[gi] = jnp.minimum(mn[gi], v)
                    mx[gi] = jnp.maximum(mx[gi], v)
                    sm[gi] = sm[gi] + v
            ob = (i // 8) % 2
            ou = i % 8
            for gi in range(ng):
                sl = pl.ds(gi * 16, 16)
                smin_v[ob, ou, sl] = mn[gi]
                smax_v[ob, ou, sl] = mx[gi]
                ssum_v[ob, ou, sl] = sm[gi]

        TB = R // 8
        d_copy(0).start()

        def body(i, carry):
            j = i - 4

            @pl.when(j >= 0)
            def _():
                @pl.when(j % 2 == 0)
                def _():
                    g_copy(j // 2).wait()

                @pl.when((j % 8 == 0) & (j >= 16))
                def _():
                    for cp in o_copies(j // 8 - 2):
                        cp.wait()

                reduce(j)

                @pl.when(j % 8 == 7)
                def _():
                    for cp in o_copies(j // 8):
                        cp.start()

            @pl.when(i < R)
            def _():
                @pl.when(i % 8 == 0)
                def _():
                    d_copy(i // 8).wait()

                    @pl.when(i // 8 + 1 < TB)
                    def _():
                        d_copy(i // 8 + 1).start()

                topk(i)

                @pl.when(i % 2 == 1)
                def _():
                    g_copy(i // 2).start()
            return carry

        lax.fori_loop(0, R + 4, body, 0)
        for tb in (TB - 2, TB - 1):
            for cp in o_copies(tb):
                cp.wait()
        pltpu.sync_copy(cnt_v, cnt_hbm.at[wid])

    return sel(d, ptf)


def _stats_body(ssum_ref, qt_ref, pt_ref, cnt_ref, o_ref):
    bb = pl.program_id(0)

    @pl.when(bb == 0)
    def _init():
        o_ref[...] = jnp.zeros_like(o_ref)

    s = ssum_ref[0]
    q = qt_ref[0]
    p = pt_ref[0]
    c = cnt_ref[0]
    o_ref[0, :] += jnp.sum(s, axis=0)
    o_ref[1, :] += jnp.sum(q, axis=0)
    o_ref[2, :] += jnp.sum(q * q, axis=0)
    o_ref[3, :] += jnp.sum(q * s, axis=0)
    o_ref[4, :] += jnp.sum(c * p * p, axis=0)


def _stats(ssum, qt, pt, cntf):
    return pl.pallas_call(
        _stats_body,
        grid=(B,),
        in_specs=[
            pl.BlockSpec((1, N, OUT), lambda b: (b, 0, 0)),
            pl.BlockSpec((1, N, OUT), lambda b: (b, 0, 0)),
            pl.BlockSpec((1, N, OUT), lambda b: (b, 0, 0)),
            pl.BlockSpec((1, N, 1), lambda b: (b, 0, 0)),
        ],
        out_specs=pl.BlockSpec((8, OUT), lambda b: (0, 0)),
        out_shape=jax.ShapeDtypeStruct((8, OUT), jnp.float32),
    )(ssum, qt, pt, cntf)


def _gelu(z):
    return 0.5 * z * (1.0 + lax.erf(z * 0.7071067811865476))


def _final_body(smin_ref, smax_ref, qt_ref, sc_ref, sh_ref, o_ref):
    q = qt_ref[0]
    sc = sc_ref[0, :]
    sh = sh_ref[0, :]
    z1 = (smin_ref[0] + q) * sc + sh
    z2 = (smax_ref[0] + q) * sc + sh
    o_ref[0] = jnp.transpose(jnp.maximum(_gelu(z1), _gelu(z2)))


def _final(smin, smax, qt, scale, shift):
    return pl.pallas_call(
        _final_body,
        grid=(B,),
        in_specs=[
            pl.BlockSpec((1, N, OUT), lambda b: (b, 0, 0)),
            pl.BlockSpec((1, N, OUT), lambda b: (b, 0, 0)),
            pl.BlockSpec((1, N, OUT), lambda b: (b, 0, 0)),
            pl.BlockSpec((1, OUT), lambda b: (0, 0)),
            pl.BlockSpec((1, OUT), lambda b: (0, 0)),
        ],
        out_specs=pl.BlockSpec((1, OUT, N), lambda b: (b, 0, 0)),
        out_shape=jax.ShapeDtypeStruct((B, OUT, N), jnp.float32),
    )(smin, smax, qt, scale, shift)


def kernel(x, W, gamma, beta):
    d, pt, qt = _prep(x, W)
    smin, smax, ssum, cnt = _select(d, pt.reshape(B * N, OUT))
    cntf = cnt.reshape(B, 2, N).sum(axis=1).astype(jnp.float32).reshape(B, N, 1)
    sums = _stats(ssum, qt, pt, cntf)
    m = float(B * N * K)
    mean = (sums[0] + K * sums[1]) / m
    ey2 = (sums[4] + 2.0 * sums[3] + K * sums[2]) / m
    var = ey2 - mean * mean
    scale = gamma / jnp.sqrt(var + EPS)
    shift = beta - mean * scale
    return _final(smin, smax, qt, scale.reshape(1, OUT), shift.reshape(1, OUT))
